# baseline (speedup 1.0000x reference)
import functools

import jax
import jax.numpy as jnp
from jax import lax
from jax.experimental import pallas as pl
from jax.experimental.pallas import tpu as pltpu
from jax.experimental.pallas import tpu_sc as plsc

_B, _T, _S, _H, _V = 4, 512, 512, 1024, 32110
_NC, _NS = 2, 16
_NW = _NC * _NS
_ROWS = _B * _T
_RPW = _ROWS // _NW
_NSG = _S // 16
_GR = 16
_PLANE = _B * _T


def _sc_body(attn_hbm, ids_hbm, bg_hbm,
             ids_v, cnt_v, vals_v, comb0_v, comb1_v, idx0_v, idx1_v, accum,
             sem_in0, sem_in1, sem_out0, sem_out1):
    c = lax.axis_index("c")
    s = lax.axis_index("s")
    wid = s * _NC + c
    b = wid // (_NW // _B)
    row0 = wid * _RPW
    t0w = row0 - b * _T

    pltpu.sync_copy(ids_hbm.at[b], ids_v)
    z16 = jnp.zeros((16,), jnp.float32)

    def zacc(i, cy):
        accum[pl.ds(i * 16, 16)] = z16
        return cy

    lax.fori_loop(0, (_V + 15) // 16, zacc, 0)

    iota16 = lax.iota(jnp.int32, 16)

    def cnt_body(kk, maxk):
        idsk = ids_v[pl.ds(kk * 16, 16)]
        cnt16 = jnp.zeros((16,), jnp.int32)
        for sh in range(1, 16):
            idx = jnp.maximum(iota16 - sh, 0)
            shifted = idsk.at[idx].get(mode="promise_in_bounds")
            eq = (idsk == shifted) & (iota16 >= sh)
            cnt16 = cnt16 + eq.astype(jnp.int32)
        cnt_v[pl.ds(kk * 16, 16)] = cnt16
        return jnp.maximum(maxk, cnt16)

    maxk = lax.fori_loop(0, _NSG, cnt_body, jnp.zeros((16,), jnp.int32))
    nrounds = jnp.max(maxk) + 1

    sems_in = (sem_in0, sem_in1)
    sems_out = (sem_out0, sem_out1)
    combs = (comb0_v, comb1_v)
    idxs = (idx0_v, idx1_v)

    def issue_load(t, half):
        pltpu.async_copy(attn_hbm.at[b, t], vals_v.at[half], sems_in[half])

    def wait_load(half):
        pltpu.make_async_copy(attn_hbm.at[b, 0], vals_v.at[half],
                              sems_in[half]).wait()

    def process_row(t, vhalf, ghalf, g):
        def round_pass(k, c2):
            def sg_body(kk4, c3):
                for u in range(4):
                    o = (kk4 * 4 + u) * 16
                    idx16 = ids_v[pl.ds(o, 16)]
                    v16 = vals_v[vhalf, pl.ds(o, 16)]
                    m16 = cnt_v[pl.ds(o, 16)] == k
                    plsc.addupdate_scatter(accum, [idx16], v16, mask=m16)
                return c3

            return lax.fori_loop(0, _NSG // 4, sg_body, c2)

        lax.fori_loop(0, nrounds, round_pass, 0)

        rowbase = ((t >> 7) * 512 + b * 128 + (t & 127)).astype(jnp.int32)

        def gath_body(kk4, c2):
            for u in range(4):
                o = (kk4 * 4 + u) * 16
                idx16 = ids_v[pl.ds(o, 16)]
                comb16 = plsc.load_gather(accum, [idx16])
                pos16 = (o + iota16) * _GR + g
                plsc.store_scatter(combs[ghalf], [pos16], comb16)
                plsc.store_scatter(idxs[ghalf], [pos16],
                                   idx16 * _PLANE + rowbase)
            return c2

        lax.fori_loop(0, _NSG // 4, gath_body, 0)

        def rezero_body(kk4, c2):
            for u in range(4):
                o = (kk4 * 4 + u) * 16
                plsc.store_scatter(accum, [ids_v[pl.ds(o, 16)]], z16)
            return c2

        lax.fori_loop(0, _NSG // 4, rezero_body, 0)

    def issue_scatters(ghalf):
        pltpu.async_copy(combs[ghalf], bg_hbm.at[idxs[ghalf]],
                         sems_out[ghalf])

    def drain_scatters(ghalf):
        pltpu.make_async_copy(bg_hbm.at[pl.ds(0, _GR * _S)],
                              combs[ghalf], sems_out[ghalf]).wait()

    issue_load(t0w, 0)
    issue_load(t0w + 1, 1)

    def gpair_body(gpi, carry):
        for ghalf in range(2):
            @pl.when(gpi >= 1)
            def _():
                drain_scatters(ghalf)

            def rowpair_body(rp, cy):
                for vhalf in range(2):
                    g = rp * 2 + vhalf
                    ridx = gpi * 2 * _GR + ghalf * _GR + g
                    t = t0w + ridx
                    wait_load(vhalf)
                    process_row(t, vhalf, ghalf, g)

                    @pl.when(ridx < _RPW - 2)
                    def _():
                        issue_load(t + 2, vhalf)

                return cy

            lax.fori_loop(0, _GR // 2, rowpair_body, 0)
            issue_scatters(ghalf)
        return carry

    lax.fori_loop(0, _RPW // (2 * _GR), gpair_body, 0)
    drain_scatters(0)
    drain_scatters(1)


_sc_scatter = functools.partial(
    pl.kernel,
    mesh=plsc.VectorSubcoreMesh(core_axis_name="c", subcore_axis_name="s",
                                num_cores=_NC, num_subcores=_NS),
    compiler_params=pltpu.CompilerParams(needs_layout_passes=False),
    scratch_types=[
        pltpu.VMEM((_S,), jnp.int32),
        pltpu.VMEM((_S,), jnp.int32),
        pltpu.VMEM((2, _S), jnp.float32),
        pltpu.VMEM((_GR * _S,), jnp.float32),
        pltpu.VMEM((_GR * _S,), jnp.float32),
        pltpu.VMEM((_GR * _S,), jnp.int32),
        pltpu.VMEM((_GR * _S,), jnp.int32),
        pltpu.VMEM((_V,), jnp.float32),
        pltpu.SemaphoreType.DMA,
        pltpu.SemaphoreType.DMA,
        pltpu.SemaphoreType.DMA,
        pltpu.SemaphoreType.DMA,
    ],
)(_sc_body)


_VT = 512
_NJ = (_V + _VT - 1) // _VT


def _zfill_body(out_ref):
    out_ref[...] = jnp.zeros((_VT, 4, _B, 128), jnp.float32)


_zfill = pl.pallas_call(
    _zfill_body,
    grid=(_NJ,),
    out_specs=pl.BlockSpec((_VT, 4, _B, 128), lambda j: (j, 0, 0, 0)),
    out_shape=jax.ShapeDtypeStruct((_V, 4, _B, 128), jnp.float32),
    compiler_params=pltpu.CompilerParams(dimension_semantics=("parallel",)),
)


def _pgen_body(dec_ref, seq_ref, w1_ref, w2_ref, b_ref, out_ref):
    d = dec_ref[...]
    q = seq_ref[...]
    acc = (jnp.sum(d * w1_ref[0][None, None, :], axis=2)
           + jnp.sum(q * w2_ref[0][None, None, :], axis=2)
           + b_ref[0, 0])
    out_ref[...] = jax.nn.sigmoid(acc)


_pgen = pl.pallas_call(
    _pgen_body,
    out_shape=jax.ShapeDtypeStruct((_B, _T), jnp.float32),
)


def kernel(decoder_input_embeds, sequence_output, cross_attentions,
           input_ids_to_copy, W, b):
    w1 = W[:_H, 0].reshape(1, _H)
    w2 = W[_H:, 0].reshape(1, _H)
    p_gen = _pgen(decoder_input_embeds, sequence_output, w1, w2,
                  b.reshape(1, 1)).reshape(_B, _T, 1)
    bg = _zfill().reshape(_V * _PLANE)
    ref = jax.new_ref(bg)
    _sc_scatter(cross_attentions, input_ids_to_copy, ref)
    phys = ref[...].reshape(_V, 4, _B, 128)
    logits = phys.transpose(2, 1, 3, 0).reshape(_B, _T, _V)
    return (p_gen, logits)


# speedup vs baseline: 16.4174x; 16.4174x over previous
import jax
import jax.numpy as jnp
from jax import lax
from jax.experimental import pallas as pl
from jax.experimental.pallas import tpu as pltpu

_B, _T, _S, _H, _V = 4, 512, 512, 1024, 32110
_VT = 1024
_NJ = (_V + _VT - 1) // _VT


def _logits_body(ids_ref, attn_t_ref, out_ref):
    j = pl.program_id(0)
    iota_v = lax.broadcasted_iota(jnp.int32, (_VT, _S), 0) + j * _VT
    for b in range(_B):
        ids_b = ids_ref[b, 0, :]
        onehot_t = (iota_v == ids_b[None, :]).astype(jnp.bfloat16)
        a_b = attn_t_ref[b]
        out_ref[:, b, :] = jnp.dot(onehot_t, a_b,
                                   preferred_element_type=jnp.float32)


_logits_t = pl.pallas_call(
    _logits_body,
    grid=(_NJ,),
    in_specs=[
        pl.BlockSpec((_B, 1, _S), lambda j: (0, 0, 0)),
        pl.BlockSpec((_B, _S, _T), lambda j: (0, 0, 0)),
    ],
    out_specs=pl.BlockSpec((_VT, _B, _T), lambda j: (j, 0, 0)),
    out_shape=jax.ShapeDtypeStruct((_V, _B, _T), jnp.float32),
    compiler_params=pltpu.CompilerParams(
        dimension_semantics=("parallel",)),
)


def _pgen_body(dec_ref, seq_ref, w1_ref, w2_ref, b_ref, out_ref):
    d = dec_ref[...]
    q = seq_ref[...]
    acc = (jnp.sum(d * w1_ref[0][None, None, :], axis=2)
           + jnp.sum(q * w2_ref[0][None, None, :], axis=2)
           + b_ref[0, 0])
    out_ref[...] = jax.nn.sigmoid(acc)


_pgen = pl.pallas_call(
    _pgen_body,
    out_shape=jax.ShapeDtypeStruct((_B, _T), jnp.float32),
)


def kernel(decoder_input_embeds, sequence_output, cross_attentions,
           input_ids_to_copy, W, b):
    w1 = W[:_H, 0].reshape(1, _H)
    w2 = W[_H:, 0].reshape(1, _H)
    p_gen = _pgen(decoder_input_embeds, sequence_output, w1, w2,
                  b.reshape(1, 1)).reshape(_B, _T, 1)
    attn_t = cross_attentions.transpose(0, 2, 1).astype(jnp.bfloat16)
    out_t = _logits_t(input_ids_to_copy.reshape(_B, 1, _S), attn_t)
    logits = out_t.transpose(1, 2, 0)
    return (p_gen, logits)
